# Initial kernel scaffold; baseline (speedup 1.0000x reference)
import jax
import jax.numpy as jnp
from jax.experimental import pallas as pl


def kernel(nodes, neighbors, features):
    raise NotImplementedError("write your pallas kernel here")


# speedup vs baseline: 2.4561x; 2.4561x over previous
import functools

import jax
import jax.numpy as jnp
from jax import lax
from jax.experimental import pallas as pl
from jax.experimental.pallas import tpu as pltpu
from jax.experimental.pallas import tpu_sc as plsc

NC, NS = 2, 16
NW = NC * NS
UNIT = 12
NSAMP = 10
DF = 128
LANES = 16


@functools.partial(jax.jit, static_argnums=(1, 2))
def _run(args, u_w, b_pad):
    features, idx = args
    mesh = plsc.VectorSubcoreMesh(core_axis_name="c", subcore_axis_name="s")

    @functools.partial(
        pl.kernel,
        mesh=mesh,
        out_type=jax.ShapeDtypeStruct((b_pad, DF), jnp.float32),
        scratch_types=[
            pltpu.VMEM((u_w, UNIT * NSAMP), jnp.int32),
            pltpu.VMEM((2 * UNIT * NSAMP, DF), jnp.float32),
            pltpu.VMEM((2 * UNIT, DF), jnp.float32),
            pltpu.SemaphoreType.DMA,
        ],
    )
    def k(feat_hbm, idx_hbm, out_hbm, idx_v, rows_v, out_v, sem):
        wid = lax.axis_index("s") * NC + lax.axis_index("c")
        pltpu.sync_copy(idx_hbm.at[wid], idx_v)
        out_base = wid * (u_w * UNIT)

        def pair(p, carry):
            cp0 = pltpu.async_copy(
                feat_hbm.at[idx_v.at[2 * p]],
                rows_v.at[pl.ds(0, UNIT * NSAMP)], sem)
            cp1 = pltpu.async_copy(
                feat_hbm.at[idx_v.at[2 * p + 1]],
                rows_v.at[pl.ds(UNIT * NSAMP, UNIT * NSAMP)], sem)
            cp0.wait()
            cp1.wait()

            def row(r, c2):
                base = r * NSAMP
                for d in range(DF // LANES):
                    sl = pl.ds(d * LANES, LANES)
                    acc = rows_v[base, sl]
                    for j in range(1, NSAMP):
                        acc = acc + rows_v[base + j, sl]
                    out_v[r, sl] = acc * jnp.float32(1.0 / NSAMP)
                return c2

            lax.fori_loop(0, 2 * UNIT, row, 0)
            pltpu.sync_copy(
                out_v, out_hbm.at[pl.ds(out_base + p * (2 * UNIT), 2 * UNIT)])
            return carry

        lax.fori_loop(0, u_w // 2, pair, 0)

    return k(features, idx)


def kernel(nodes, neighbors, features):
    del nodes
    b = neighbors.shape[0]
    per_w = -(-b // NW)
    u_w = -(-per_w // UNIT)
    u_w += u_w % 2
    b_pad = NW * u_w * UNIT
    flat = neighbors.reshape(-1)
    pad = b_pad * NSAMP - flat.shape[0]
    if pad:
        flat = jnp.concatenate([flat, jnp.zeros((pad,), jnp.int32)])
    idx = flat.reshape(NW, u_w, UNIT * NSAMP)
    out = _run((features, idx), u_w, b_pad)
    return out[:b]


# speedup vs baseline: 3.1203x; 1.2704x over previous
import functools

import jax
import jax.numpy as jnp
from jax import lax
from jax.experimental import pallas as pl
from jax.experimental.pallas import tpu as pltpu
from jax.experimental.pallas import tpu_sc as plsc

NC, NS = 2, 16
NW = NC * NS
UNIT = 12
NSAMP = 10
DF = 128
LANES = 16
HALF = UNIT * NSAMP
PROWS = 2 * UNIT


@functools.partial(jax.jit, static_argnums=(1, 2))
def _run(args, u_w, b_pad):
    features, idx = args
    mesh = plsc.VectorSubcoreMesh(core_axis_name="c", subcore_axis_name="s")
    npairs = u_w // 2

    @functools.partial(
        pl.kernel,
        mesh=mesh,
        out_type=jax.ShapeDtypeStruct((b_pad, DF), jnp.float32),
        scratch_types=[
            pltpu.VMEM((u_w, HALF), jnp.int32),
            pltpu.VMEM((2 * HALF, DF), jnp.float32),
            pltpu.VMEM((2 * HALF, DF), jnp.float32),
            pltpu.VMEM((PROWS, DF), jnp.float32),
            pltpu.VMEM((PROWS, DF), jnp.float32),
            pltpu.SemaphoreType.DMA,
            pltpu.SemaphoreType.DMA,
            pltpu.SemaphoreType.DMA,
            pltpu.SemaphoreType.DMA,
        ],
    )
    def k(feat_hbm, idx_hbm, out_hbm, idx_v, rows0, rows1, out0, out1,
          gsem0, gsem1, osem0, osem1):
        wid = lax.axis_index("s") * NC + lax.axis_index("c")
        pltpu.sync_copy(idx_hbm.at[wid], idx_v)
        out_base = wid * (u_w * UNIT)

        def fire(p, rows, s):
            pltpu.async_copy(
                feat_hbm.at[idx_v.at[2 * p]], rows.at[pl.ds(0, HALF)], s)
            pltpu.async_copy(
                feat_hbm.at[idx_v.at[2 * p + 1]],
                rows.at[pl.ds(HALF, HALF)], s)

        def drain(p, rows, s):
            pltpu.make_async_copy(
                feat_hbm.at[idx_v.at[2 * p]],
                rows.at[pl.ds(0, HALF)], s).wait()
            pltpu.make_async_copy(
                feat_hbm.at[idx_v.at[2 * p + 1]],
                rows.at[pl.ds(HALF, HALF)], s).wait()

        def out_slice(p):
            return out_hbm.at[pl.ds(out_base + p * PROWS, PROWS)]

        def reduce(rows, out):
            def row(r, c2):
                base = r * NSAMP
                for d in range(DF // LANES):
                    sl = pl.ds(d * LANES, LANES)
                    v = [rows[base + j, sl] for j in range(NSAMP)]
                    while len(v) > 1:
                        v = [a + b for a, b in zip(v[::2], v[1::2])] + (
                            [v[-1]] if len(v) % 2 else [])
                    out[r, sl] = v[0] * jnp.float32(1.0 / NSAMP)
                return c2

            lax.fori_loop(0, PROWS, row, 0)

        fire(0, rows0, gsem0)

        def step(t, carry):
            p0 = 2 * t
            p1 = 2 * t + 1
            fire(p1, rows1, gsem1)
            drain(p0, rows0, gsem0)

            @pl.when(t > 0)
            def _():
                pltpu.make_async_copy(out0, out_slice(p0), osem0).wait()

            reduce(rows0, out0)
            pltpu.async_copy(out0, out_slice(p0), osem0)

            @pl.when(t + 1 < npairs // 2)
            def _():
                fire(p0 + 2, rows0, gsem0)

            drain(p1, rows1, gsem1)

            @pl.when(t > 0)
            def _():
                pltpu.make_async_copy(out1, out_slice(p1), osem1).wait()

            reduce(rows1, out1)
            pltpu.async_copy(out1, out_slice(p1), osem1)
            return carry

        lax.fori_loop(0, npairs // 2, step, 0)
        pltpu.make_async_copy(out0, out_slice(npairs - 2), osem0).wait()
        pltpu.make_async_copy(out1, out_slice(npairs - 1), osem1).wait()

    return k(features, idx)


def kernel(nodes, neighbors, features):
    del nodes
    b = neighbors.shape[0]
    per_w = -(-b // NW)
    u_w = -(-per_w // UNIT)
    u_w += (-u_w) % 4
    b_pad = NW * u_w * UNIT
    flat = neighbors.reshape(-1)
    pad = b_pad * NSAMP - flat.shape[0]
    if pad:
        flat = jnp.concatenate([flat, jnp.zeros((pad,), jnp.int32)])
    idx = flat.reshape(NW, u_w, HALF)
    out = _run((features, idx), u_w, b_pad)
    return out[:b]


# speedup vs baseline: 3.4763x; 1.1141x over previous
import functools

import jax
import jax.numpy as jnp
from jax import lax
from jax.experimental import pallas as pl
from jax.experimental.pallas import tpu as pltpu
from jax.experimental.pallas import tpu_sc as plsc

NC, NS = 2, 16
NW = NC * NS
UNIT = 12
NSAMP = 10
DF = 128
LANES = 16
HALF = UNIT * NSAMP
PROWS = 2 * UNIT
FRAC_C0 = 0.74


@functools.partial(jax.jit, static_argnums=(1, 2, 3))
def _run(args, n0, n1, b_pad):
    features, idx = args
    u_max = max(n0, n1)
    mesh = plsc.VectorSubcoreMesh(core_axis_name="c", subcore_axis_name="s")

    @functools.partial(
        pl.kernel,
        mesh=mesh,
        out_type=jax.ShapeDtypeStruct((b_pad, DF), jnp.float32),
        scratch_types=[
            pltpu.VMEM((u_max, HALF), jnp.int32),
            pltpu.VMEM((2 * HALF, DF), jnp.float32),
            pltpu.VMEM((2 * HALF, DF), jnp.float32),
            pltpu.VMEM((PROWS, DF), jnp.float32),
            pltpu.VMEM((PROWS, DF), jnp.float32),
            pltpu.SemaphoreType.DMA,
            pltpu.SemaphoreType.DMA,
            pltpu.SemaphoreType.DMA,
            pltpu.SemaphoreType.DMA,
        ],
    )
    def k(feat_hbm, idx_hbm, out_hbm, idx_v, rows0, rows1, out0, out1,
          gsem0, gsem1, osem0, osem1):
        c = lax.axis_index("c")
        s = lax.axis_index("s")
        wid = s * NC + c
        pltpu.sync_copy(idx_hbm.at[wid], idx_v)
        cnt = jnp.where(c == 0, n0, n1)
        nsteps = cnt // 4
        npairs = cnt // 2
        out_base = (s * (n0 + n1) + c * n0) * UNIT

        def fire(p, rows, sem):
            pltpu.async_copy(
                feat_hbm.at[idx_v.at[2 * p]], rows.at[pl.ds(0, HALF)], sem)
            pltpu.async_copy(
                feat_hbm.at[idx_v.at[2 * p + 1]],
                rows.at[pl.ds(HALF, HALF)], sem)

        def drain(p, rows, sem):
            pltpu.make_async_copy(
                feat_hbm.at[idx_v.at[2 * p]],
                rows.at[pl.ds(0, HALF)], sem).wait()
            pltpu.make_async_copy(
                feat_hbm.at[idx_v.at[2 * p + 1]],
                rows.at[pl.ds(HALF, HALF)], sem).wait()

        def out_slice(p):
            return out_hbm.at[pl.ds(out_base + p * PROWS, PROWS)]

        def reduce(rows, out):
            def row(r, c2):
                base = r * NSAMP
                for d in range(DF // LANES):
                    sl = pl.ds(d * LANES, LANES)
                    v = [rows[base + j, sl] for j in range(NSAMP)]
                    while len(v) > 1:
                        v = [a + b for a, b in zip(v[::2], v[1::2])] + (
                            [v[-1]] if len(v) % 2 else [])
                    out[r, sl] = v[0] * jnp.float32(1.0 / NSAMP)
                return c2

            lax.fori_loop(0, PROWS, row, 0)

        fire(0, rows0, gsem0)

        def step(t, carry):
            p0 = 2 * t
            p1 = 2 * t + 1
            fire(p1, rows1, gsem1)
            drain(p0, rows0, gsem0)

            @pl.when(t > 0)
            def _():
                pltpu.make_async_copy(out0, out_slice(p0), osem0).wait()

            reduce(rows0, out0)
            pltpu.async_copy(out0, out_slice(p0), osem0)

            @pl.when(t + 1 < nsteps)
            def _():
                fire(p0 + 2, rows0, gsem0)

            drain(p1, rows1, gsem1)

            @pl.when(t > 0)
            def _():
                pltpu.make_async_copy(out1, out_slice(p1), osem1).wait()

            reduce(rows1, out1)
            pltpu.async_copy(out1, out_slice(p1), osem1)
            return carry

        lax.fori_loop(0, nsteps, step, 0)
        pltpu.make_async_copy(out0, out_slice(npairs - 2), osem0).wait()
        pltpu.make_async_copy(out1, out_slice(npairs - 1), osem1).wait()

    return k(features, idx)


def _split(total_units):
    n0 = int(round(total_units * FRAC_C0 / 4.0)) * 4
    n0 = min(max(n0, 4), total_units - 4)
    return n0, total_units - n0


def kernel(nodes, neighbors, features):
    del nodes
    b = neighbors.shape[0]
    u_total = -(-b // UNIT)
    per_s = -(-u_total // NS)
    per_s += (-per_s) % 8
    n0, n1 = _split(per_s)
    b_pad = NS * per_s * UNIT
    flat = neighbors.reshape(-1)
    pad = b_pad * NSAMP - flat.shape[0]
    if pad:
        flat = jnp.concatenate([flat, jnp.zeros((pad,), jnp.int32)])
    units = flat.reshape(NS, per_s, HALF)
    u_max = max(n0, n1)
    w0 = units[:, :n0, :]
    w1 = units[:, n0:, :]
    if n0 < u_max:
        w0 = jnp.pad(w0, ((0, 0), (0, u_max - n0), (0, 0)))
    if n1 < u_max:
        w1 = jnp.pad(w1, ((0, 0), (0, u_max - n1), (0, 0)))
    idx = jnp.stack([w0, w1], axis=1).reshape(NW, u_max, HALF)
    out = _run((features, idx), n0, n1, b_pad)
    return out[:b]


# speedup vs baseline: 3.7624x; 1.0823x over previous
import functools

import jax
import jax.numpy as jnp
from jax import lax
from jax.experimental import pallas as pl
from jax.experimental.pallas import tpu as pltpu
from jax.experimental.pallas import tpu_sc as plsc

NC, NS = 2, 16
NW = NC * NS
UNIT = 12
NSAMP = 10
DF = 128
LANES = 16
HALF = UNIT * NSAMP
PROWS = 2 * UNIT
FRAC_C0 = 0.85


@functools.partial(jax.jit, static_argnums=(1, 2, 3))
def _run(args, n0, n1, b_pad):
    features, idx = args
    u_max = max(n0, n1)
    mesh = plsc.VectorSubcoreMesh(core_axis_name="c", subcore_axis_name="s")

    @functools.partial(
        pl.kernel,
        mesh=mesh,
        out_type=jax.ShapeDtypeStruct((b_pad, DF), jnp.float32),
        scratch_types=[
            pltpu.VMEM((u_max, HALF), jnp.int32),
            pltpu.VMEM((2 * HALF, DF), jnp.float32),
            pltpu.VMEM((2 * HALF, DF), jnp.float32),
            pltpu.VMEM((PROWS, DF), jnp.float32),
            pltpu.VMEM((PROWS, DF), jnp.float32),
            pltpu.SemaphoreType.DMA,
            pltpu.SemaphoreType.DMA,
            pltpu.SemaphoreType.DMA,
            pltpu.SemaphoreType.DMA,
        ],
    )
    def k(feat_hbm, idx_hbm, out_hbm, idx_v, rows0, rows1, out0, out1,
          gsem0, gsem1, osem0, osem1):
        c = lax.axis_index("c")
        s = lax.axis_index("s")
        wid = s * NC + c
        pltpu.sync_copy(idx_hbm.at[wid], idx_v)
        cnt = jnp.where(c == 0, n0, n1)
        nsteps = cnt // 4
        npairs = cnt // 2
        out_base = (s * (n0 + n1) + c * n0) * UNIT

        def fire(p, rows, sem):
            pltpu.async_copy(
                feat_hbm.at[idx_v.at[2 * p]], rows.at[pl.ds(0, HALF)], sem)
            pltpu.async_copy(
                feat_hbm.at[idx_v.at[2 * p + 1]],
                rows.at[pl.ds(HALF, HALF)], sem)

        def drain(p, rows, sem):
            pltpu.make_async_copy(
                feat_hbm.at[idx_v.at[2 * p]],
                rows.at[pl.ds(0, HALF)], sem).wait()
            pltpu.make_async_copy(
                feat_hbm.at[idx_v.at[2 * p + 1]],
                rows.at[pl.ds(HALF, HALF)], sem).wait()

        def out_slice(p):
            return out_hbm.at[pl.ds(out_base + p * PROWS, PROWS)]

        def reduce(rows, out):
            def row(r, c2):
                base = r * NSAMP
                for d in range(DF // LANES):
                    sl = pl.ds(d * LANES, LANES)
                    v = [rows[base + j, sl] for j in range(NSAMP)]
                    while len(v) > 1:
                        v = [a + b for a, b in zip(v[::2], v[1::2])] + (
                            [v[-1]] if len(v) % 2 else [])
                    out[r, sl] = v[0] * jnp.float32(1.0 / NSAMP)
                return c2

            lax.fori_loop(0, PROWS, row, 0)

        fire(0, rows0, gsem0)

        def step(t, carry):
            p0 = 2 * t
            p1 = 2 * t + 1
            fire(p1, rows1, gsem1)
            drain(p0, rows0, gsem0)

            @pl.when(t > 0)
            def _():
                pltpu.make_async_copy(out0, out_slice(p0), osem0).wait()

            reduce(rows0, out0)
            pltpu.async_copy(out0, out_slice(p0), osem0)

            @pl.when(t + 1 < nsteps)
            def _():
                fire(p0 + 2, rows0, gsem0)

            drain(p1, rows1, gsem1)

            @pl.when(t > 0)
            def _():
                pltpu.make_async_copy(out1, out_slice(p1), osem1).wait()

            reduce(rows1, out1)
            pltpu.async_copy(out1, out_slice(p1), osem1)
            return carry

        lax.fori_loop(0, nsteps, step, 0)
        pltpu.make_async_copy(out0, out_slice(npairs - 2), osem0).wait()
        pltpu.make_async_copy(out1, out_slice(npairs - 1), osem1).wait()

    return k(features, idx)


def _split(total_units):
    n0 = int(round(total_units * FRAC_C0 / 4.0)) * 4
    n0 = min(max(n0, 4), total_units - 4)
    return n0, total_units - n0


def kernel(nodes, neighbors, features):
    del nodes
    b = neighbors.shape[0]
    u_total = -(-b // UNIT)
    per_s = -(-u_total // NS)
    per_s += (-per_s) % 8
    n0, n1 = _split(per_s)
    b_pad = NS * per_s * UNIT
    flat = neighbors.reshape(-1)
    pad = b_pad * NSAMP - flat.shape[0]
    if pad:
        flat = jnp.concatenate([flat, jnp.zeros((pad,), jnp.int32)])
    units = flat.reshape(NS, per_s, HALF)
    u_max = max(n0, n1)
    w0 = units[:, :n0, :]
    w1 = units[:, n0:, :]
    if n0 < u_max:
        w0 = jnp.pad(w0, ((0, 0), (0, u_max - n0), (0, 0)))
    if n1 < u_max:
        w1 = jnp.pad(w1, ((0, 0), (0, u_max - n1), (0, 0)))
    idx = jnp.stack([w0, w1], axis=1).reshape(NW, u_max, HALF)
    out = _run((features, idx), n0, n1, b_pad)
    return out[:b]
